# baseline (speedup 1.0000x reference)
import jax
import jax.numpy as jnp
from jax import lax
from jax.experimental import pallas as pl
from jax.experimental.pallas import tpu as pltpu
from jax.experimental.pallas import tpu_sc as plsc

_D = 64
_ROPE_BASE = 10000.0
_NC = 2
_NS = 16
_NW = _NC * _NS
_BPW = 128
_PAD = 128


def _tec_body(x_hbm, table_hbm, trig_hbm, out_hbm,
              x_slab, idx_v, trig_v, rows0, rows1, ob0, ob1,
              sg0, sg1, sw0, sw1):
    seq = x_hbm.shape[1]
    wid = lax.axis_index("s") * _NC + lax.axis_index("c")
    b0 = wid * _BPW

    pltpu.sync_copy(x_hbm.at[pl.ds(b0, _BPW)], x_slab)
    pltpu.sync_copy(trig_hbm, trig_v)
    tbl = table_hbm

    row_ids = [g * 16 + lax.iota(jnp.int32, 16) for g in range(_BPW // 16)]

    @plsc.parallel_loop(0, seq, 1, unroll=2)
    def _transpose_col(l):
        col = jnp.full((16,), l, jnp.int32)
        for g in range(_BPW // 16):
            idx_v[l, pl.ds(g * 16, 16)] = plsc.load_gather(
                x_slab, [row_ids[g], col])

    rows = (rows0, rows1)
    ob = (ob0, ob1)
    sg = (sg0, sg1)
    sw = (sw0, sw1)

    pltpu.async_copy(tbl.at[idx_v.at[0]], rows0, sg0)

    def step(l2, carry):
        for p in range(2):
            l = l2 * 2 + p

            @pl.when(l + 1 < seq)
            def _fire_next():
                pltpu.async_copy(tbl.at[idx_v.at[l + 1]], rows[1 - p],
                                 sg[1 - p])

            pltpu.make_async_copy(tbl.at[idx_v.at[l]], rows[p],
                                  sg[p]).wait()

            @pl.when(l >= 2)
            def _drain_write():
                pltpu.make_async_copy(ob[p], out_hbm.at[pl.ds(b0, _BPW), l],
                                      sw[p]).wait()

            c0 = trig_v[l, pl.ds(0, 16)]
            c1 = trig_v[l, pl.ds(16, 16)]
            s0 = trig_v[l, pl.ds(32, 16)]
            s1 = trig_v[l, pl.ds(48, 16)]
            ns0 = trig_v[l, pl.ds(64, 16)]
            ns1 = trig_v[l, pl.ds(80, 16)]

            rp = rows[p]
            op = ob[p]

            @plsc.parallel_loop(0, _BPW, 1, unroll=8)
            def _rope_row(r):
                h0 = rp[r, pl.ds(0, 16)]
                h1 = rp[r, pl.ds(16, 16)]
                h2 = rp[r, pl.ds(32, 16)]
                h3 = rp[r, pl.ds(48, 16)]
                op[r, pl.ds(0, 16)] = h0 * c0 + h2 * ns0
                op[r, pl.ds(16, 16)] = h1 * c1 + h3 * ns1
                op[r, pl.ds(32, 16)] = h2 * c0 + h0 * s0
                op[r, pl.ds(48, 16)] = h3 * c1 + h1 * s1

            pltpu.async_copy(ob[p], out_hbm.at[pl.ds(b0, _BPW), l], sw[p])
        return carry

    lax.fori_loop(0, seq // 2, step, 0)

    pltpu.make_async_copy(ob[0], out_hbm.at[pl.ds(b0, _BPW), 0], sw[0]).wait()
    pltpu.make_async_copy(ob[1], out_hbm.at[pl.ds(b0, _BPW), 1], sw[1]).wait()


def kernel(x, emb_table, pos_table):
    del pos_table
    b, l = x.shape
    idx = x.astype(jnp.int32)

    half = _D // 2
    fi = jnp.arange(half, dtype=jnp.float32)
    freqs = 1.0 / (_ROPE_BASE ** (fi / half))
    ang = jnp.arange(l, dtype=jnp.float32)[:, None] * freqs[None, :]
    cos, sin = jnp.cos(ang), jnp.sin(ang)
    trig = jnp.concatenate(
        [cos, sin, -sin, jnp.zeros((l, half), jnp.float32)], axis=1)

    mesh = plsc.VectorSubcoreMesh(core_axis_name="c", subcore_axis_name="s")
    out = pl.kernel(
        _tec_body,
        out_type=jax.ShapeDtypeStruct((b, l, _D), jnp.float32),
        mesh=mesh,
        compiler_params=pltpu.CompilerParams(use_tc_tiling_on_sc=False, needs_layout_passes=False),
        scratch_types=[
            pltpu.VMEM((_BPW, l), jnp.int32),
            pltpu.VMEM((l, _BPW), jnp.int32),
            pltpu.VMEM((l, _PAD), jnp.float32),
            pltpu.VMEM((_BPW, _D), jnp.float32),
            pltpu.VMEM((_BPW, _D), jnp.float32),
            pltpu.VMEM((_BPW, _D), jnp.float32),
            pltpu.VMEM((_BPW, _D), jnp.float32),
            pltpu.SemaphoreType.DMA,
            pltpu.SemaphoreType.DMA,
            pltpu.SemaphoreType.DMA,
            pltpu.SemaphoreType.DMA,
        ],
    )(idx, emb_table, trig)
    return out


# speedup vs baseline: 1.0630x; 1.0630x over previous
import jax
import jax.numpy as jnp
from jax import lax
from jax.experimental import pallas as pl
from jax.experimental.pallas import tpu as pltpu
from jax.experimental.pallas import tpu_sc as plsc

_D = 64
_ROPE_BASE = 10000.0
_NC = 2
_NS = 16
_NW = _NC * _NS
_BPW = 128
_PAD = 128

def _rope_body(x_hbm, table_hbm, trig_hbm, out_hbm,
               x_slab, ib, trig_v, rows0, rows1, ob0, ob1,
               sg0, sg1, sw0, sw1):
    seq = trig_hbm.shape[0]
    wid = lax.axis_index("s") * _NC + lax.axis_index("c")
    b0 = wid * _BPW

    pltpu.sync_copy(x_hbm.at[pl.ds(b0 * seq, _BPW * seq)],
                    x_slab.at[pl.ds(0, _BPW * seq)])
    pltpu.sync_copy(trig_hbm, trig_v)

    i200 = [lax.iota(jnp.int32, 16) * seq + g * 16 * seq
            for g in range(_BPW // 16)]

    def _build_indices(l, j):
        for g in range(_BPW // 16):
            tok = plsc.load_gather(x_slab, [i200[g] + l])
            ib[j, pl.ds(g * 16, 16)] = lax.shift_right_logical(tok, 1)

    rows = (rows0, rows1)
    ob = (ob0, ob1)
    sg = (sg0, sg1)
    sw = (sw0, sw1)

    _build_indices(0, 0)
    pltpu.async_copy(table_hbm.at[ib.at[0]], rows0, sg0)

    def step(l2, carry):
        for p in range(2):
            l = l2 * 2 + p

            @pl.when(l + 1 < seq)
            def _fire_next():
                _build_indices(l + 1, 1 - p)
                pltpu.async_copy(table_hbm.at[ib.at[1 - p]], rows[1 - p],
                                 sg[1 - p])

            pltpu.make_async_copy(table_hbm.at[ib.at[p]], rows[p],
                                  sg[p]).wait()

            @pl.when(l >= 2)
            def _drain_write():
                pltpu.make_async_copy(ob[p], out_hbm.at[pl.ds(b0, _BPW), l],
                                      sw[p]).wait()

            c0 = trig_v[l, pl.ds(0, 16)]
            c1 = trig_v[l, pl.ds(16, 16)]
            s0 = trig_v[l, pl.ds(32, 16)]
            s1 = trig_v[l, pl.ds(48, 16)]
            ns0 = trig_v[l, pl.ds(64, 16)]
            ns1 = trig_v[l, pl.ds(80, 16)]

            rp = rows[p]
            op = ob[p]

            @plsc.parallel_loop(0, _BPW, 1, unroll=8)
            def _rope_row(r):
                tok = x_slab[pl.ds(r * seq + l, 16)][0]
                off = lax.shift_left(tok & 1, 6)
                h0 = rp[r, pl.ds(off, 16)]
                h1 = rp[r, pl.ds(off + 16, 16)]
                h2 = rp[r, pl.ds(off + 32, 16)]
                h3 = rp[r, pl.ds(off + 48, 16)]
                op[r, pl.ds(0, 16)] = h0 * c0 + h2 * ns0
                op[r, pl.ds(16, 16)] = h1 * c1 + h3 * ns1
                op[r, pl.ds(32, 16)] = h2 * c0 + h0 * s0
                op[r, pl.ds(48, 16)] = h3 * c1 + h1 * s1

            pltpu.async_copy(ob[p], out_hbm.at[pl.ds(b0, _BPW), l], sw[p])
        return carry

    lax.fori_loop(0, seq // 2, step, 0)

    pltpu.make_async_copy(ob[0], out_hbm.at[pl.ds(b0, _BPW), 0], sw[0]).wait()
    pltpu.make_async_copy(ob[1], out_hbm.at[pl.ds(b0, _BPW), 1], sw[1]).wait()


def kernel(x, emb_table, pos_table):
    del pos_table
    b, l = x.shape
    v = emb_table.shape[0]
    mesh = plsc.VectorSubcoreMesh(core_axis_name="c", subcore_axis_name="s")

    table_p = emb_table.reshape(v // 2, _PAD)

    idx = x.reshape(b * l).astype(jnp.int32)

    half = _D // 2
    fi = jnp.arange(half, dtype=jnp.float32)
    freqs = 1.0 / (_ROPE_BASE ** (fi / half))
    ang = jnp.arange(l, dtype=jnp.float32)[:, None] * freqs[None, :]
    cos, sin = jnp.cos(ang), jnp.sin(ang)
    trig = jnp.concatenate(
        [cos, sin, -sin, jnp.zeros((l, half), jnp.float32)], axis=1)

    out = pl.kernel(
        _rope_body,
        out_type=jax.ShapeDtypeStruct((b, l, _D), jnp.float32),
        mesh=mesh,
        compiler_params=pltpu.CompilerParams(needs_layout_passes=False),
        scratch_types=[
            pltpu.VMEM((_BPW * l + 16,), jnp.int32),
            pltpu.VMEM((2, _BPW), jnp.int32),
            pltpu.VMEM((l, _PAD), jnp.float32),
            pltpu.VMEM((_BPW, _PAD), jnp.float32),
            pltpu.VMEM((_BPW, _PAD), jnp.float32),
            pltpu.VMEM((_BPW, _D), jnp.float32),
            pltpu.VMEM((_BPW, _D), jnp.float32),
            pltpu.SemaphoreType.DMA,
            pltpu.SemaphoreType.DMA,
            pltpu.SemaphoreType.DMA,
            pltpu.SemaphoreType.DMA,
        ],
    )(idx, table_p, trig)
    return out


# speedup vs baseline: 1.0636x; 1.0005x over previous
import jax
import jax.numpy as jnp
from jax import lax
from jax.experimental import pallas as pl
from jax.experimental.pallas import tpu as pltpu
from jax.experimental.pallas import tpu_sc as plsc

_D = 64
_ROPE_BASE = 10000.0
_NC = 2
_NS = 16
_NW = _NC * _NS
_BPW = 128
_PAD = 128

_RB = 8000


def _pack_tc_body(t_ref, o_ref):
    ev = t_ref[pl.Slice(0, _RB // 2, 2), :]
    od = t_ref[pl.Slice(1, _RB // 2, 2), :]
    o_ref[...] = jnp.concatenate([ev, od], axis=1)


def _rope_body(x_hbm, table_hbm, trig_hbm, out_hbm,
               tb0, tb1, ib0, ib1, trig_v, rows0, rows1, ob0, ob1,
               sg0, sg1, sw0, sw1):
    seq = trig_hbm.shape[0]
    wid = lax.axis_index("s") * _NC + lax.axis_index("c")
    b0 = wid * _BPW
    g1 = (seq // 2 + 7) // 8 * 8
    g2 = seq - g1

    pltpu.sync_copy(trig_hbm, trig_v)

    rows = (rows0, rows1)
    ob = (ob0, ob1)
    tb = (tb0, tb1)
    ib = (ib0, ib1)
    sg = (sg0, sg1)
    sw = (sw0, sw1)

    def _stage(c, j):
        pltpu.sync_copy(x_hbm.at[pl.ds((b0 + c) * seq, seq)],
                        tb[j].at[pl.ds(0, seq)])
        for g in range((seq + 15) // 16):
            o = min(g * 16, seq - 16)
            tok = tb[j][pl.ds(o, 16)]
            ib[j][pl.ds(o, 16)] = lax.shift_right_logical(tok, 1)
        pltpu.async_copy(table_hbm.at[ib[j].at[pl.ds(0, g1)]],
                         rows[j].at[pl.ds(0, g1)], sg[j])
        pltpu.async_copy(table_hbm.at[ib[j].at[pl.ds(g1, g2)]],
                         rows[j].at[pl.ds(g1, g2)], sg[j])

    _stage(0, 0)

    def step(c2, carry):
        for p in range(2):
            c = c2 * 2 + p

            @pl.when(c + 1 < _BPW)
            def _fire_next():
                _stage(c + 1, 1 - p)

            pltpu.make_async_copy(table_hbm.at[ib[p].at[pl.ds(0, g1)]],
                                  rows[p].at[pl.ds(0, g1)], sg[p]).wait()
            pltpu.make_async_copy(table_hbm.at[ib[p].at[pl.ds(g1, g2)]],
                                  rows[p].at[pl.ds(g1, g2)], sg[p]).wait()

            @pl.when(c >= 2)
            def _drain_write():
                pltpu.make_async_copy(ob[p], out_hbm.at[b0 + c], sw[p]).wait()

            rp = rows[p]
            op = ob[p]
            tp = tb[p]

            @plsc.parallel_loop(0, seq, 1, unroll=8)
            def _rope_row(r):
                tok = tp[pl.ds(r, 16)][0]
                off = lax.shift_left(tok & 1, 6)
                c0 = trig_v[r, pl.ds(0, 16)]
                c1 = trig_v[r, pl.ds(16, 16)]
                s0 = trig_v[r, pl.ds(32, 16)]
                s1 = trig_v[r, pl.ds(48, 16)]
                ns0 = trig_v[r, pl.ds(64, 16)]
                ns1 = trig_v[r, pl.ds(80, 16)]
                h0 = rp[r, pl.ds(off, 16)]
                h1 = rp[r, pl.ds(off + 16, 16)]
                h2 = rp[r, pl.ds(off + 32, 16)]
                h3 = rp[r, pl.ds(off + 48, 16)]
                op[r, pl.ds(0, 16)] = h0 * c0 + h2 * ns0
                op[r, pl.ds(16, 16)] = h1 * c1 + h3 * ns1
                op[r, pl.ds(32, 16)] = h2 * c0 + h0 * s0
                op[r, pl.ds(48, 16)] = h3 * c1 + h1 * s1

            pltpu.async_copy(ob[p], out_hbm.at[b0 + c], sw[p])
        return carry

    lax.fori_loop(0, _BPW // 2, step, 0)

    pltpu.make_async_copy(ob[0], out_hbm.at[b0], sw[0]).wait()
    pltpu.make_async_copy(ob[1], out_hbm.at[b0 + 1], sw[1]).wait()


def kernel(x, emb_table, pos_table):
    del pos_table
    b, l = x.shape
    v = emb_table.shape[0]

    table_p = pl.pallas_call(
        _pack_tc_body,
        grid=(v // _RB,),
        in_specs=[pl.BlockSpec((_RB, _D), lambda i: (i, 0))],
        out_specs=pl.BlockSpec((_RB // 2, _PAD), lambda i: (i, 0)),
        out_shape=jax.ShapeDtypeStruct((v // 2, _PAD), jnp.float32),
    )(emb_table)

    idx = x.reshape(b * l).astype(jnp.int32)

    half = _D // 2
    fi = jnp.arange(half, dtype=jnp.float32)
    freqs = 1.0 / (_ROPE_BASE ** (fi / half))
    ang = jnp.arange(l, dtype=jnp.float32)[:, None] * freqs[None, :]
    cos, sin = jnp.cos(ang), jnp.sin(ang)
    trig = jnp.concatenate(
        [cos, sin, -sin, jnp.zeros((l, half), jnp.float32)], axis=1)

    mesh = plsc.VectorSubcoreMesh(core_axis_name="c", subcore_axis_name="s")
    out = pl.kernel(
        _rope_body,
        out_type=jax.ShapeDtypeStruct((b, l, _D), jnp.float32),
        mesh=mesh,
        scratch_types=[
            pltpu.VMEM((l + 16,), jnp.int32),
            pltpu.VMEM((l + 16,), jnp.int32),
            pltpu.VMEM((l,), jnp.int32),
            pltpu.VMEM((l,), jnp.int32),
            pltpu.VMEM((l, _PAD), jnp.float32),
            pltpu.VMEM((l, _PAD), jnp.float32),
            pltpu.VMEM((l, _PAD), jnp.float32),
            pltpu.VMEM((l, _D), jnp.float32),
            pltpu.VMEM((l, _D), jnp.float32),
            pltpu.SemaphoreType.DMA,
            pltpu.SemaphoreType.DMA,
            pltpu.SemaphoreType.DMA,
            pltpu.SemaphoreType.DMA,
        ],
    )(idx, table_p, trig)
    return out
